# Initial kernel scaffold; baseline (speedup 1.0000x reference)
import jax
import jax.numpy as jnp
from jax.experimental import pallas as pl


def kernel(x):
    raise NotImplementedError("write your pallas kernel here")


# speedup vs baseline: 1.2725x; 1.2725x over previous
import jax
import jax.numpy as jnp
from jax import lax
from jax.experimental import pallas as pl
from jax.experimental.pallas import tpu as pltpu
from jax.experimental.pallas import tpu_sc as plsc

L = 16
R, C = 64, 32768
NC, NS = 2, 16
NW = NC * NS
RPW = R // NW
CHUNKS = C // L
UNROLL = 8

_NEG = float("-inf")


def _tec_body(x_hbm, out_hbm, xv, resv):
    cid = lax.axis_index("c")
    sid = lax.axis_index("s")
    wid = sid * NC + cid
    base = wid * RPW
    pltpu.sync_copy(x_hbm.at[pl.ds(base * C, RPW * C)], xv)
    lane = lax.iota(jnp.int32, L)
    for r in range(RPW):
        def step(i, carry):
            t0, t1, t2 = carry
            off = r * C + i * (L * UNROLL)
            for j in range(UNROLL):
                v = xv[pl.ds(off + j * L, L)]
                lo = jnp.minimum(t0, v)
                t0 = jnp.maximum(t0, v)
                lo2 = jnp.minimum(t1, lo)
                t1 = jnp.maximum(t1, lo)
                t2 = jnp.maximum(t2, lo2)
            return t0, t1, t2

        full = jnp.full((L,), _NEG, jnp.float32)
        t0, t1, t2 = lax.fori_loop(0, CHUNKS // UNROLL, step, (full, full, full))

        def pop(t0, t1, t2):
            m = jnp.max(t0)
            j = plsc.all_reduce_ffs(t0 == m)
            sel = lane == j
            return (m, jnp.where(sel, t1, t0), jnp.where(sel, t2, t1),
                    jnp.where(sel, _NEG, t2))

        m1, t0, t1, t2 = pop(t0, t1, t2)
        m2, t0, t1, t2 = pop(t0, t1, t2)
        m3 = jnp.max(t0)
        res = jnp.where(lane == 0, m1,
                        jnp.where(lane == 1, m2,
                                  jnp.where(lane == 2, m3, jnp.float32(0.0))))
        resv[pl.ds(r * L, L)] = res
    pltpu.sync_copy(resv, out_hbm.at[pl.ds(base * L, RPW * L)])


def kernel(x):
    mesh = plsc.VectorSubcoreMesh(core_axis_name="c", subcore_axis_name="s")
    f = pl.kernel(
        _tec_body,
        mesh=mesh,
        compiler_params=pltpu.CompilerParams(needs_layout_passes=False),
        out_type=jax.ShapeDtypeStruct((R * L,), jnp.float32),
        scratch_types=[
            pltpu.VMEM((RPW * C,), jnp.float32),
            pltpu.VMEM((RPW * L,), jnp.float32),
        ],
    )
    out = f(x.reshape(R * C))
    return out.reshape(R, L)[:, :3]
